# baseline (speedup 1.0000x reference)
import functools

import jax
import jax.numpy as jnp
from jax import lax
from jax.experimental import pallas as pl
from jax.experimental.pallas import tpu as pltpu
from jax.experimental.pallas import tpu_sc as plsc

B = 4096
L = 200
D = 32
OUT_DIM = 20

NC = 2
NS = 16
NW = NC * NS
BPW = B // NW
IPW = BPW * L
C0 = 104
C1 = L - C0
NBUF = 8

VOCAB_PAD = 1 << 20
QV = VOCAB_PAD // 4
PROJ_BLK = 8192

_mesh = plsc.VectorSubcoreMesh(core_axis_name="c", subcore_axis_name="s")


def _proj_body(t0_ref, t1_ref, t2_ref, t3_ref, wpad_ref, p_ref):
    w = wpad_ref[...].astype(jnp.bfloat16)
    dn = (((0,), (1,)), ((), ()))

    def dot(t_ref):
        return lax.dot_general(
            t_ref[...].astype(jnp.bfloat16), w, dn,
            preferred_element_type=jnp.float32,
        )

    p_ref[...] = jnp.concatenate(
        [dot(t0_ref), dot(t1_ref), dot(t2_ref), dot(t3_ref)], axis=1
    )


def _tspec(u):
    nblk = QV // PROJ_BLK
    last = 1000000 // PROJ_BLK
    return pl.BlockSpec(
        (D, PROJ_BLK),
        lambda i, u=u: (0, jnp.minimum(u * nblk + i, last)),
    )


_project = pl.pallas_call(
    _proj_body,
    grid=(QV // PROJ_BLK,),
    in_specs=[
        _tspec(0), _tspec(1), _tspec(2), _tspec(3),
        pl.BlockSpec((D, D), lambda i: (0, 0)),
    ],
    out_specs=pl.BlockSpec((PROJ_BLK, 128), lambda i: (i, 0)),
    out_shape=jax.ShapeDtypeStruct((QV, 128), jnp.float32),
)


@functools.partial(
    pl.kernel,
    out_type=jax.ShapeDtypeStruct((B, D), jnp.float32),
    mesh=_mesh,
    scratch_types=[
        pltpu.VMEM((BPW, L), jnp.int32),
        pltpu.VMEM((BPW, D), jnp.float32),
        pltpu.VMEM((NBUF, L, D), jnp.float32),
        [pltpu.SemaphoreType.DMA] * NBUF,
    ],
    compiler_params=pltpu.CompilerParams(use_tc_tiling_on_sc=False),
)
def _pool(data_hbm, table_hbm, out_hbm, idx_v, out_v, rows_v, sems):
    wid = lax.axis_index("s") * NC + lax.axis_index("c")
    base = wid * BPW

    pltpu.sync_copy(data_hbm.at[pl.ds(base, BPW)], idx_v)

    def fire(i, slot):
        rows = rows_v.at[slot]
        pltpu.async_copy(
            table_hbm.at[idx_v.at[i, pl.ds(0, C0)]], rows.at[pl.ds(0, C0)],
            sems[slot],
        )
        pltpu.async_copy(
            table_hbm.at[idx_v.at[i, pl.ds(C0, C1)]], rows.at[pl.ds(C0, C1)],
            sems[slot],
        )

    def drain(i, slot):
        rows = rows_v.at[slot]
        pltpu.make_async_copy(
            table_hbm.at[idx_v.at[i, pl.ds(0, C0)]], rows.at[pl.ds(0, C0)],
            sems[slot],
        ).wait()
        pltpu.make_async_copy(
            table_hbm.at[idx_v.at[i, pl.ds(C0, C1)]], rows.at[pl.ds(C0, C1)],
            sems[slot],
        ).wait()

    def accumulate(i, slot):
        rows = rows_v.at[slot]
        z = jnp.zeros((16,), jnp.float32)
        a = [z] * 2
        bb = [z] * 2
        for l in range(L):
            c = l % 2
            a[c] = a[c] + rows[l, pl.ds(0, 16)]
            bb[c] = bb[c] + rows[l, pl.ds(16, 16)]
        out_v[i, pl.ds(0, 16)] = a[0] + a[1]
        out_v[i, pl.ds(16, 16)] = bb[0] + bb[1]

    for s in range(NBUF):
        fire(s, s)

    def loop_body(j, _):
        i = j * NBUF
        for s in range(NBUF):
            drain(i + s, s)
            accumulate(i + s, s)

            @pl.when(i + s + NBUF < BPW)
            def _refire(i=i, s=s):
                fire(i + s + NBUF, s)

        return 0

    lax.fori_loop(0, BPW // NBUF, loop_body, 0)

    pltpu.sync_copy(out_v, out_hbm.at[pl.ds(base, BPW)])


def _epilogue_body(pooled_ref, len_ref, b_ref, out_ref):
    x = pooled_ref[...][:, :OUT_DIM] / len_ref[...].astype(jnp.float32)
    out_ref[...] = x + b_ref[...]


_epilogue = pl.pallas_call(
    _epilogue_body,
    out_shape=jax.ShapeDtypeStruct((B, OUT_DIM), jnp.float32),
)


def kernel(data, length, table, W, b):
    data = data.astype(jnp.int32)
    tableT = jnp.swapaxes(table, 0, 1)
    wpad = jnp.zeros((D, D), jnp.float32).at[:OUT_DIM].set(W)
    proj = _project(tableT, tableT, tableT, tableT, wpad)
    proj_rows = proj.reshape(VOCAB_PAD, D)
    data = ((data & (QV - 1)) << 2) | (data >> 18)
    pooled = _pool(data, proj_rows)
    return _epilogue(pooled, length.reshape(B, 1), b.reshape(1, OUT_DIM))


# speedup vs baseline: 1.0580x; 1.0580x over previous
import functools

import jax
import jax.numpy as jnp
from jax import lax
from jax.experimental import pallas as pl
from jax.experimental.pallas import tpu as pltpu
from jax.experimental.pallas import tpu_sc as plsc

B = 4096
L = 200
D = 32
OUT_DIM = 20

NC = 2
NS = 16
NW = NC * NS
BPW = B // NW
IPW = BPW * L
C0 = 104
C1 = L - C0
NBUF = 4

VOCAB_PAD = 1 << 20
QV = VOCAB_PAD // 4
PROJ_BLK = 8192

_mesh = plsc.VectorSubcoreMesh(core_axis_name="c", subcore_axis_name="s")


def _proj_body(t0_ref, t1_ref, t2_ref, t3_ref, wpad_ref, p_ref):
    w = wpad_ref[...].astype(jnp.bfloat16)
    dn = (((0,), (1,)), ((), ()))

    def dot(t_ref):
        return lax.dot_general(
            t_ref[...].astype(jnp.bfloat16), w, dn,
            preferred_element_type=jnp.float32,
        )

    p_ref[...] = jnp.concatenate(
        [dot(t0_ref), dot(t1_ref), dot(t2_ref), dot(t3_ref)], axis=1
    )


def _tspec(u):
    nblk = QV // PROJ_BLK
    last = 1000000 // PROJ_BLK
    return pl.BlockSpec(
        (D, PROJ_BLK),
        lambda i, u=u: (0, jnp.minimum(u * nblk + i, last)),
    )


_project = pl.pallas_call(
    _proj_body,
    grid=(QV // PROJ_BLK,),
    in_specs=[
        _tspec(0), _tspec(1), _tspec(2), _tspec(3),
        pl.BlockSpec((D, D), lambda i: (0, 0)),
    ],
    out_specs=pl.BlockSpec((PROJ_BLK, 128), lambda i: (i, 0)),
    out_shape=jax.ShapeDtypeStruct((QV, 128), jnp.float32),
)


@functools.partial(
    pl.kernel,
    out_type=jax.ShapeDtypeStruct((B, D), jnp.float32),
    mesh=_mesh,
    scratch_types=[
        pltpu.VMEM((BPW, L), jnp.int32),
        pltpu.VMEM((BPW, D), jnp.float32),
        pltpu.VMEM((NBUF, L, D), jnp.float32),
        [pltpu.SemaphoreType.DMA] * NBUF,
    ],
    compiler_params=pltpu.CompilerParams(use_tc_tiling_on_sc=False),
)
def _pool(data_hbm, table_hbm, out_hbm, idx_v, out_v, rows_v, sems):
    wid = lax.axis_index("s") * NC + lax.axis_index("c")
    base = wid * BPW

    pltpu.sync_copy(data_hbm.at[pl.ds(base, BPW)], idx_v)

    def fire(i, slot):
        rows = rows_v.at[slot]
        pltpu.async_copy(
            table_hbm.at[idx_v.at[i, pl.ds(0, C0)]], rows.at[pl.ds(0, C0)],
            sems[slot],
        )
        pltpu.async_copy(
            table_hbm.at[idx_v.at[i, pl.ds(C0, C1)]], rows.at[pl.ds(C0, C1)],
            sems[slot],
        )

    def drain(i, slot):
        rows = rows_v.at[slot]
        pltpu.make_async_copy(
            table_hbm.at[idx_v.at[i, pl.ds(0, C0)]], rows.at[pl.ds(0, C0)],
            sems[slot],
        ).wait()
        pltpu.make_async_copy(
            table_hbm.at[idx_v.at[i, pl.ds(C0, C1)]], rows.at[pl.ds(C0, C1)],
            sems[slot],
        ).wait()

    def accumulate(i, slot):
        rows = rows_v.at[slot]
        z = jnp.zeros((16,), jnp.float32)
        a = [z] * 2
        bb = [z] * 2
        for l in range(L):
            c = l % 2
            a[c] = a[c] + rows[l, pl.ds(0, 16)]
            bb[c] = bb[c] + rows[l, pl.ds(16, 16)]
        out_v[i, pl.ds(0, 16)] = a[0] + a[1]
        out_v[i, pl.ds(16, 16)] = bb[0] + bb[1]

    for s in range(NBUF):
        fire(s, s)

    def loop_body(j, _):
        i = j * NBUF
        for s in range(NBUF):
            drain(i + s, s)
            accumulate(i + s, s)

            @pl.when(i + s + NBUF < BPW)
            def _refire(i=i, s=s):
                fire(i + s + NBUF, s)

        return 0

    lax.fori_loop(0, BPW // NBUF, loop_body, 0)

    pltpu.sync_copy(out_v, out_hbm.at[pl.ds(base, BPW)])


def _epilogue_body(pooled_ref, len_ref, b_ref, out_ref):
    x = pooled_ref[...][:, :OUT_DIM] / len_ref[...].astype(jnp.float32)
    out_ref[...] = x + b_ref[...]


_epilogue = pl.pallas_call(
    _epilogue_body,
    out_shape=jax.ShapeDtypeStruct((B, OUT_DIM), jnp.float32),
)


def kernel(data, length, table, W, b):
    data = data.astype(jnp.int32)
    tableT = jnp.swapaxes(table, 0, 1)
    wpad = jnp.zeros((D, D), jnp.float32).at[:OUT_DIM].set(W)
    proj = _project(tableT, tableT, tableT, tableT, wpad)
    proj_rows = proj.reshape(VOCAB_PAD, D)
    data = ((data & (QV - 1)) << 2) | (data >> 18)
    pooled = _pool(data, proj_rows)
    return _epilogue(pooled, length.reshape(B, 1), b.reshape(1, OUT_DIM))
